# baseline (speedup 1.0000x reference)
import jax
import jax.numpy as jnp
from jax import lax
from jax.experimental import pallas as pl
from jax.experimental.pallas import tpu as pltpu
from jax.experimental.pallas import tpu_sc as plsc

B = 1024
L = 200
DW = 300
DP = 5
DOUT = DW + 2 * DP
N = B * L
V = 100002
VP = 100096
VT = VP // 128
FT = 304 // 8
NC = 2
NS = 16
NW = NC * NS
CN = 3200
NCH = N // CN
LANES = 16
NPLANES = 10
NSH = NCH


def _gather_plane_chunk(ibuf, obuf, colbuf):
    @plsc.parallel_loop(0, CN // LANES, unroll=16)
    def _grp(g):
        sl = pl.ds(g * LANES, LANES)
        v = ibuf[sl]
        vals = plsc.load_gather(
            colbuf, [lax.shift_right_logical(v, 7), lax.bitwise_and(v, 127)])
        obuf[sl] = vals


def _gather_pos_chunk(ibuf, obuf, ptab, j):
    @plsc.parallel_loop(0, CN // LANES, unroll=16)
    def _grp(g):
        sl = pl.ds(g * LANES, LANES)
        pi = ibuf[sl]
        vals = plsc.load_gather(ptab, [pi * DP + j])
        obuf[sl] = vals


def _embed_kernel(widx_hbm, p1i_hbm, p2i_hbm, wt4_hbm, p1t_hbm, p2t_hbm,
                  out_hbm,
                  col_v, ib0, ib1, ob0, ob1, p1t_v, p2t_v, widx_sh,
                  sem_i0, sem_i1, sem_o0, sem_o1):
    sid = lax.axis_index("s")
    w = sid * NC + lax.axis_index("c")

    pltpu.sync_copy(p1t_hbm, p1t_v)
    pltpu.sync_copy(p2t_hbm, p2t_v)

    shsl = pl.ds(sid * (N // NS), N // NS)
    pltpu.sync_copy(widx_hbm.at[shsl], widx_sh.at[shsl])
    plsc.subcore_barrier()

    def plane(k, carry):
        f = jnp.where(k < 9, w + 32 * k, 288 + w)
        is_word = f < DW

        @pl.when(f < DOUT)
        def _run_plane():
            @pl.when(is_word)
            def _load_col():
                fg = lax.shift_right_logical(f, 3)
                fi = lax.bitwise_and(f, 7)
                pltpu.sync_copy(wt4_hbm.at[fg, :, fi, :], col_v)

            j1 = f - DW
            j2 = f - DW - DP
            is_p1 = jnp.logical_and(f >= DW, f < DW + DP)

            ibufs = (ib0, ib1)
            obufs = (ob0, ob1)
            isems = (sem_i0, sem_i1)
            osems = (sem_o0, sem_o1)

            def load_idx(c, buf, sem):
                src = pl.ds(c * CN, CN)

                @pl.when(is_word)
                def _():
                    pltpu.async_copy(widx_sh.at[src], buf, sem)

                @pl.when(is_p1)
                def _():
                    pltpu.async_copy(p1i_hbm.at[src], buf, sem)

                @pl.when(jnp.logical_and(jnp.logical_not(is_word),
                                         jnp.logical_not(is_p1)))
                def _():
                    pltpu.async_copy(p2i_hbm.at[src], buf, sem)

            load_idx(0, ib0, sem_i0)

            def do_chunk(c, ib, ob, isem, osem, tguard):
                pltpu.make_async_copy(widx_sh.at[pl.ds(0, CN)], ib,
                                      isem).wait()

                @pl.when(tguard)
                def _():
                    pltpu.make_async_copy(ob, out_hbm.at[0, pl.ds(0, CN)],
                                          osem).wait()

                @pl.when(is_word)
                def _():
                    _gather_plane_chunk(ib, ob, col_v)

                @pl.when(is_p1)
                def _():
                    _gather_pos_chunk(ib, ob, p1t_v, j1)

                @pl.when(jnp.logical_and(jnp.logical_not(is_word),
                                         jnp.logical_not(is_p1)))
                def _():
                    _gather_pos_chunk(ib, ob, p2t_v, j2)

                pltpu.async_copy(ob, out_hbm.at[f, pl.ds(c * CN, CN)], osem)

            def pair(t, cc):
                c0 = 2 * t
                load_idx(c0 + 1, ib1, sem_i1)
                do_chunk(c0, ib0, ob0, sem_i0, sem_o0, t > 0)

                @pl.when(t < NCH // 2 - 1)
                def _():
                    load_idx(c0 + 2, ib0, sem_i0)

                do_chunk(c0 + 1, ib1, ob1, sem_i1, sem_o1, t > 0)
                return cc

            lax.fori_loop(0, NCH // 2, pair, 0)
            pltpu.make_async_copy(ob0, out_hbm.at[0, pl.ds(0, CN)],
                                  sem_o0).wait()
            pltpu.make_async_copy(ob1, out_hbm.at[0, pl.ds(0, CN)],
                                  sem_o1).wait()

        return carry

    lax.fori_loop(0, NPLANES, plane, 0)


def kernel(word, posh, post, word_table, pos1_table, pos2_table):
    widx_t = (word.astype(jnp.int32).reshape(8, 128, 25, 8)
              .transpose(2, 0, 3, 1).reshape(N))
    p1i_t = (posh.astype(jnp.int32).reshape(8, 128, 25, 8)
             .transpose(2, 0, 3, 1).reshape(N))
    p2i_t = (post.astype(jnp.int32).reshape(8, 128, 25, 8)
             .transpose(2, 0, 3, 1).reshape(N))
    wt4 = (jnp.pad(word_table, ((0, VP - V), (0, 304 - DW)))
           .reshape(VT, 128, FT, 8).transpose(2, 0, 3, 1))
    p1_f = pos1_table.reshape(2 * L * DP)
    p2_f = pos2_table.reshape(2 * L * DP)

    run = pl.kernel(
        _embed_kernel,
        out_type=jax.ShapeDtypeStruct((DOUT, N), jnp.float32),
        mesh=plsc.VectorSubcoreMesh(core_axis_name="c", subcore_axis_name="s"),
        scratch_types=[
            pltpu.VMEM((VT, 128), jnp.float32),
            pltpu.VMEM((CN,), jnp.int32),
            pltpu.VMEM((CN,), jnp.int32),
            pltpu.VMEM((CN,), jnp.float32),
            pltpu.VMEM((CN,), jnp.float32),
            pltpu.VMEM((2 * L * DP,), jnp.float32),
            pltpu.VMEM((2 * L * DP,), jnp.float32),
            pltpu.VMEM_SHARED((NSH * CN,), jnp.int32),
            pltpu.SemaphoreType.DMA,
            pltpu.SemaphoreType.DMA,
            pltpu.SemaphoreType.DMA,
            pltpu.SemaphoreType.DMA,
        ],
        compiler_params=pltpu.CompilerParams(use_tc_tiling_on_sc=False,
                                             needs_layout_passes=False),
    )
    r = run(widx_t, p1i_t, p2i_t, wt4, p1_f, p2_f)
    return (r.reshape(DOUT, 25, 8, 8, 128).transpose(2, 4, 1, 3, 0)
            .reshape(B, L, DOUT))


# speedup vs baseline: 1.1128x; 1.1128x over previous
import jax
import jax.numpy as jnp
from jax import lax
from jax.experimental import pallas as pl
from jax.experimental.pallas import tpu as pltpu
from jax.experimental.pallas import tpu_sc as plsc

B = 1024
L = 200
DW = 300
DP = 5
DOUT = DW + 2 * DP
N = B * L
V = 100002
VP = 100096
VT = VP // 128
FT = 304 // 8
NC = 2
NS = 16
NW = NC * NS
CN = 3200
NCH = N // CN
LANES = 16
NPLANES = 10
NSH = NCH


def _gather_plane_chunk(ibuf, obuf, colbuf):
    @plsc.parallel_loop(0, CN // LANES, unroll=8)
    def _grp(g):
        sl = pl.ds(g * LANES, LANES)
        v = ibuf[sl]
        vals = plsc.load_gather(
            colbuf, [lax.shift_right_logical(v, 7), lax.bitwise_and(v, 127)])
        obuf[sl] = vals


def _gather_pos_chunk(ibuf, obuf, ptab, j):
    @plsc.parallel_loop(0, CN // LANES, unroll=8)
    def _grp(g):
        sl = pl.ds(g * LANES, LANES)
        pi = ibuf[sl]
        vals = plsc.load_gather(ptab, [pi * DP + j])
        obuf[sl] = vals


def _embed_kernel(widx_hbm, p1i_hbm, p2i_hbm, wt4_hbm, p1t_hbm, p2t_hbm,
                  out_hbm,
                  col_v, ib0, ib1, ob0, ob1, p1t_v, p2t_v, widx_sh,
                  sem_i0, sem_i1, sem_o0, sem_o1):
    sid = lax.axis_index("s")
    w = sid * NC + lax.axis_index("c")

    pltpu.sync_copy(p1t_hbm, p1t_v)
    pltpu.sync_copy(p2t_hbm, p2t_v)

    shsl = pl.ds(sid * (N // NS), N // NS)
    pltpu.sync_copy(widx_hbm.at[shsl], widx_sh.at[shsl])
    plsc.subcore_barrier()

    def plane(k, carry):
        f = jnp.where(k < 9, w + 32 * k, 288 + w)
        is_word = f < DW

        @pl.when(f < DOUT)
        def _run_plane():
            @pl.when(is_word)
            def _load_col():
                fg = lax.shift_right_logical(f, 3)
                fi = lax.bitwise_and(f, 7)
                pltpu.sync_copy(wt4_hbm.at[fg, :, fi, :], col_v)

            j1 = f - DW
            j2 = f - DW - DP
            is_p1 = jnp.logical_and(f >= DW, f < DW + DP)

            ibufs = (ib0, ib1)
            obufs = (ob0, ob1)
            isems = (sem_i0, sem_i1)
            osems = (sem_o0, sem_o1)

            def load_idx(c, buf, sem):
                src = pl.ds(c * CN, CN)

                @pl.when(is_word)
                def _():
                    pltpu.async_copy(widx_sh.at[src], buf, sem)

                @pl.when(is_p1)
                def _():
                    pltpu.async_copy(p1i_hbm.at[src], buf, sem)

                @pl.when(jnp.logical_and(jnp.logical_not(is_word),
                                         jnp.logical_not(is_p1)))
                def _():
                    pltpu.async_copy(p2i_hbm.at[src], buf, sem)

            load_idx(0, ib0, sem_i0)

            def do_chunk(c, ib, ob, isem, osem, tguard):
                pltpu.make_async_copy(widx_sh.at[pl.ds(0, CN)], ib,
                                      isem).wait()

                @pl.when(tguard)
                def _():
                    pltpu.make_async_copy(ob, out_hbm.at[0, pl.ds(0, CN)],
                                          osem).wait()

                @pl.when(is_word)
                def _():
                    _gather_plane_chunk(ib, ob, col_v)

                @pl.when(is_p1)
                def _():
                    _gather_pos_chunk(ib, ob, p1t_v, j1)

                @pl.when(jnp.logical_and(jnp.logical_not(is_word),
                                         jnp.logical_not(is_p1)))
                def _():
                    _gather_pos_chunk(ib, ob, p2t_v, j2)

                pltpu.async_copy(ob, out_hbm.at[f, pl.ds(c * CN, CN)], osem)

            def pair(t, cc):
                c0 = 2 * t
                load_idx(c0 + 1, ib1, sem_i1)
                do_chunk(c0, ib0, ob0, sem_i0, sem_o0, t > 0)

                @pl.when(t < NCH // 2 - 1)
                def _():
                    load_idx(c0 + 2, ib0, sem_i0)

                do_chunk(c0 + 1, ib1, ob1, sem_i1, sem_o1, t > 0)
                return cc

            lax.fori_loop(0, NCH // 2, pair, 0)
            pltpu.make_async_copy(ob0, out_hbm.at[0, pl.ds(0, CN)],
                                  sem_o0).wait()
            pltpu.make_async_copy(ob1, out_hbm.at[0, pl.ds(0, CN)],
                                  sem_o1).wait()

        return carry

    lax.fori_loop(0, NPLANES, plane, 0)


def kernel(word, posh, post, word_table, pos1_table, pos2_table):
    widx_t = (word.astype(jnp.int32).reshape(8, 128, 25, 8)
              .transpose(2, 0, 3, 1).reshape(N))
    p1i_t = (posh.astype(jnp.int32).reshape(8, 128, 25, 8)
             .transpose(2, 0, 3, 1).reshape(N))
    p2i_t = (post.astype(jnp.int32).reshape(8, 128, 25, 8)
             .transpose(2, 0, 3, 1).reshape(N))
    wt4 = (jnp.pad(word_table, ((0, VP - V), (0, 304 - DW)))
           .reshape(VT, 128, FT, 8).transpose(2, 0, 3, 1))
    p1_f = pos1_table.reshape(2 * L * DP)
    p2_f = pos2_table.reshape(2 * L * DP)

    run = pl.kernel(
        _embed_kernel,
        out_type=jax.ShapeDtypeStruct((DOUT, N), jnp.float32),
        mesh=plsc.VectorSubcoreMesh(core_axis_name="c", subcore_axis_name="s"),
        scratch_types=[
            pltpu.VMEM((VT, 128), jnp.float32),
            pltpu.VMEM((CN,), jnp.int32),
            pltpu.VMEM((CN,), jnp.int32),
            pltpu.VMEM((CN,), jnp.float32),
            pltpu.VMEM((CN,), jnp.float32),
            pltpu.VMEM((2 * L * DP,), jnp.float32),
            pltpu.VMEM((2 * L * DP,), jnp.float32),
            pltpu.VMEM_SHARED((NSH * CN,), jnp.int32),
            pltpu.SemaphoreType.DMA,
            pltpu.SemaphoreType.DMA,
            pltpu.SemaphoreType.DMA,
            pltpu.SemaphoreType.DMA,
        ],
        compiler_params=pltpu.CompilerParams(use_tc_tiling_on_sc=False,
                                             needs_layout_passes=False),
    )
    r = run(widx_t, p1i_t, p2i_t, wt4, p1_f, p2_f)
    return (r.reshape(DOUT, 25, 8, 8, 128).transpose(2, 4, 1, 3, 0)
            .reshape(B, L, DOUT))
